# baseline (speedup 1.0000x reference)
import functools

import jax
import jax.numpy as jnp
from jax import lax
from jax.experimental import pallas as pl
from jax.experimental.pallas import tpu as pltpu
from jax.experimental.pallas import tpu_sc as plsc

_B, _P, _E, _V, _NT = 1024, 20, 64, 100, 33
_N = _B * _P
_NR = _N * _NT
_NW = 32
_BPW = _NR // _NW
_CH = 128
_EP = 128
_NCH = _BPW // _CH
_RB = 640
_H = 256


def _sc_gather_make():
    mesh = plsc.VectorSubcoreMesh(core_axis_name="c", subcore_axis_name="s")

    @functools.partial(
        pl.kernel,
        mesh=mesh,
        out_type=jax.ShapeDtypeStruct((_NR, _EP), jnp.float32),
        scratch_types=[
            pltpu.VMEM((_BPW,), jnp.int32),
            pltpu.VMEM((_CH, _EP), jnp.float32),
            pltpu.SemaphoreType.DMA,
        ],
    )
    def sc_gather(tbl_hbm, idx_hbm, out_hbm, idx_v, rows_v, sem):
        wid = lax.axis_index("s") * 2 + lax.axis_index("c")
        base = wid * _BPW
        pltpu.sync_copy(idx_hbm.at[pl.ds(base, _BPW)], idx_v)

        @pl.loop(0, _NCH)
        def _(i):
            pltpu.async_copy(
                tbl_hbm.at[idx_v.at[pl.ds(i * _CH, _CH)]], rows_v, sem
            ).wait()
            pltpu.sync_copy(rows_v, out_hbm.at[pl.ds(base + i * _CH, _CH)])

    return sc_gather


def _ffn_kernel(emb_ref, xd_ref, w1_ref, b1_ref, w2_ref, o_ref):
    lane = lax.broadcasted_iota(jnp.int32, (_RB, 2 * _E), 1)
    d01 = jnp.where(lane < _E, xd_ref[:, 0:1], xd_ref[:, 1:2]).astype(jnp.bfloat16)
    cat = jnp.concatenate([d01, emb_ref[...].astype(jnp.bfloat16)], axis=1)
    acc = jax.lax.dot(cat, w1_ref[...],
                      precision=jax.lax.Precision.DEFAULT,
                      preferred_element_type=jnp.float32)
    h = jnp.maximum(acc + b1_ref[...], 0.0)
    o_ref[...] = jax.lax.dot(h.astype(jnp.bfloat16), w2_ref[...],
                             precision=jax.lax.Precision.DEFAULT,
                             preferred_element_type=jnp.float32)


def _softmax_kernel(l_ref, o_ref):
    l = l_ref[...]
    m = jnp.max(l, axis=1, keepdims=True)
    e = jnp.exp(l - m)
    o_ref[...] = e / jnp.sum(e, axis=1, keepdims=True)


def kernel(x, tables, W1, b1, W2, b2):
    idx = x[:, :, 2:].astype(jnp.int32).reshape(_N, _NT)
    fidx = (idx + jnp.arange(_NT, dtype=jnp.int32)[None, :] * _V).reshape(_NR)
    xd = x[:, :, :2].reshape(_N, 2)
    tbl2d = jnp.pad(tables.reshape(_NT * _V, _E), ((0, 0), (0, _EP - _E)))

    gathered = _sc_gather_make()(tbl2d, fidx)
    emb2d = gathered.reshape(_N, _NT * _EP)

    w1t = W1.T
    w1f = jnp.pad(w1t[2 * _E :].reshape(_NT, _E, _H),
                  ((0, 0), (0, _EP - _E), (0, 0))).reshape(_NT * _EP, _H)
    w1bf = jnp.concatenate([w1t[: 2 * _E], w1f]).astype(jnp.bfloat16)
    w2bf = W2.T.astype(jnp.bfloat16)

    logits = pl.pallas_call(
        _ffn_kernel,
        grid=(_N // _RB,),
        in_specs=[
            pl.BlockSpec((_RB, _NT * _EP), lambda i: (i, 0)),
            pl.BlockSpec((_RB, 2), lambda i: (i, 0)),
            pl.BlockSpec((2 * _E + _NT * _EP, _H), lambda i: (0, 0)),
            pl.BlockSpec((1, _H), lambda i: (0, 0)),
            pl.BlockSpec((_H, 1), lambda i: (0, 0)),
        ],
        out_specs=pl.BlockSpec((_RB, 1), lambda i: (i, 0)),
        out_shape=jax.ShapeDtypeStruct((_N, 1), jnp.float32),
    )(emb2d, xd, w1bf, b1.reshape(1, _H), w2bf)

    out = pl.pallas_call(
        _softmax_kernel,
        out_shape=jax.ShapeDtypeStruct((_B, _P), jnp.float32),
    )(logits.reshape(_B, _P))

    return out.reshape(_B, _P, 1)


# speedup vs baseline: 5.7698x; 5.7698x over previous
import functools

import jax
import jax.numpy as jnp
from jax.experimental import pallas as pl
from jax.experimental.pallas import tpu as pltpu

_B, _P, _E, _V, _NT = 1024, 20, 64, 100, 33
_NF = _NT + 2
_VP = 112
_R = 640
_H = 256
_K = _NT * _VP


def _proj_kernel(w_ref, t_ref, o_ref):
    o_ref[0] = jax.lax.dot(
        w_ref[0], t_ref[0],
        precision=jax.lax.Precision.DEFAULT,
        preferred_element_type=jnp.float32,
    )


def _ffn_kernel(idx_ref, xd_ref, tb_ref, u_ref, b1_ref, w2_ref, o_ref, oh_ref):
    sub = jax.lax.broadcasted_iota(jnp.int32, (_VP, _R), 0)
    for f in range(_NT):
        oh = (sub == idx_ref[f : f + 1, :]).astype(jnp.bfloat16)
        oh_ref[f * _VP : (f + 1) * _VP, :] = oh
    acc = jax.lax.dot(
        tb_ref[...], oh_ref[...],
        precision=jax.lax.Precision.DEFAULT,
        preferred_element_type=jnp.float32,
    )
    acc = acc + u_ref[:, 0:1] * xd_ref[0:1, :] + u_ref[:, 1:2] * xd_ref[1:2, :]
    h = jnp.maximum(acc + b1_ref[...], 0.0)
    o_ref[0] = jax.lax.dot(
        w2_ref[...], h.astype(jnp.bfloat16),
        precision=jax.lax.Precision.DEFAULT,
        preferred_element_type=jnp.float32,
    )


def _softmax_kernel(l_ref, o_ref):
    l = l_ref[...]
    m = jnp.max(l, axis=1, keepdims=True)
    e = jnp.exp(l - m)
    o_ref[...] = e / jnp.sum(e, axis=1, keepdims=True)


def _logits(x, tables, W1, b1, W2, b2):
    n = _B * _P
    idx_t = x[:, :, 2:].astype(jnp.int32).reshape(n, _NT).T
    xd_t = x[:, :, :2].reshape(n, 2).T

    tt = jnp.zeros((_NF, _E, _VP), jnp.float32)
    tt = tt.at[0:2].set(1.0)
    tt = tt.at[2:, :, :_V].set(tables.transpose(0, 2, 1))
    w1t = W1.reshape(_H, _NF, _E).transpose(1, 0, 2)

    proj = pl.pallas_call(
        _proj_kernel,
        grid=(_NF,),
        in_specs=[
            pl.BlockSpec((1, _H, _E), lambda f: (f, 0, 0)),
            pl.BlockSpec((1, _E, _VP), lambda f: (f, 0, 0)),
        ],
        out_specs=pl.BlockSpec((1, _H, _VP), lambda f: (f, 0, 0)),
        out_shape=jax.ShapeDtypeStruct((_NF, _H, _VP), jnp.float32),
    )(w1t.astype(jnp.bfloat16), tt.astype(jnp.bfloat16))

    u01 = proj[0:2, :, 0].T
    tb = proj[2:].transpose(1, 0, 2).reshape(_H, _K).astype(jnp.bfloat16)
    w2b = W2.astype(jnp.bfloat16)

    nblk = n // _R
    logits = pl.pallas_call(
        _ffn_kernel,
        grid=(nblk,),
        in_specs=[
            pl.BlockSpec((_NT, _R), lambda i: (0, i)),
            pl.BlockSpec((2, _R), lambda i: (0, i)),
            pl.BlockSpec((_H, _K), lambda i: (0, 0)),
            pl.BlockSpec((_H, 2), lambda i: (0, 0)),
            pl.BlockSpec((_H, 1), lambda i: (0, 0)),
            pl.BlockSpec((1, _H), lambda i: (0, 0)),
        ],
        out_specs=pl.BlockSpec((1, 1, _R), lambda i: (i, 0, 0)),
        out_shape=jax.ShapeDtypeStruct((nblk, 1, _R), jnp.float32),
        scratch_shapes=[pltpu.VMEM((_K, _R), jnp.bfloat16)],
    )(idx_t, xd_t, tb, u01, b1.reshape(_H, 1), w2b)
    return logits


def kernel(x, tables, W1, b1, W2, b2):
    logits = _logits(x, tables, W1, b1, W2, b2)
    out = pl.pallas_call(
        _softmax_kernel,
        out_shape=jax.ShapeDtypeStruct((_B, _P), jnp.float32),
    )(logits.reshape(_B, _P))

    return out.reshape(_B, _P, 1)


# speedup vs baseline: 6.4042x; 1.1100x over previous
import functools

import jax
import jax.numpy as jnp
from jax.experimental import pallas as pl
from jax.experimental.pallas import tpu as pltpu

_B, _P, _E, _V, _NT = 1024, 20, 64, 100, 33
_NF = _NT + 2
_VP = 112
_R = 1280
_H = 256
_K = _NT * _VP


def _proj_kernel(w_ref, t_ref, o_ref):
    o_ref[0] = jax.lax.dot(
        w_ref[0], t_ref[0],
        precision=jax.lax.Precision.DEFAULT,
        preferred_element_type=jnp.float32,
    )


def _ffn_kernel(idx_ref, xd_ref, tb_ref, u_ref, b1_ref, w2_ref, o_ref, oh_ref):
    sub = jax.lax.broadcasted_iota(jnp.int32, (_VP, _R), 0)
    for f in range(_NT):
        oh = (sub == idx_ref[f : f + 1, :]).astype(jnp.bfloat16)
        oh_ref[f * _VP : (f + 1) * _VP, :] = oh
    acc = jax.lax.dot(
        tb_ref[...], oh_ref[...],
        precision=jax.lax.Precision.DEFAULT,
        preferred_element_type=jnp.float32,
    )
    acc = acc + u_ref[:, 0:1] * xd_ref[0:1, :] + u_ref[:, 1:2] * xd_ref[1:2, :]
    h = jnp.maximum(acc + b1_ref[...], 0.0)
    o_ref[0] = jax.lax.dot(
        w2_ref[...], h.astype(jnp.bfloat16),
        precision=jax.lax.Precision.DEFAULT,
        preferred_element_type=jnp.float32,
    )


def _softmax_kernel(l_ref, o_ref):
    l = l_ref[...]
    m = jnp.max(l, axis=1, keepdims=True)
    e = jnp.exp(l - m)
    o_ref[...] = e / jnp.sum(e, axis=1, keepdims=True)


def _logits(x, tables, W1, b1, W2, b2):
    n = _B * _P
    idx_t = x[:, :, 2:].astype(jnp.int32).reshape(n, _NT).T
    xd_t = x[:, :, :2].reshape(n, 2).T

    tt = jnp.zeros((_NF, _E, _VP), jnp.float32)
    tt = tt.at[0:2].set(1.0)
    tt = tt.at[2:, :, :_V].set(tables.transpose(0, 2, 1))
    w1t = W1.reshape(_H, _NF, _E).transpose(1, 0, 2)

    proj = pl.pallas_call(
        _proj_kernel,
        grid=(_NF,),
        in_specs=[
            pl.BlockSpec((1, _H, _E), lambda f: (f, 0, 0)),
            pl.BlockSpec((1, _E, _VP), lambda f: (f, 0, 0)),
        ],
        out_specs=pl.BlockSpec((1, _H, _VP), lambda f: (f, 0, 0)),
        out_shape=jax.ShapeDtypeStruct((_NF, _H, _VP), jnp.float32),
    )(w1t.astype(jnp.bfloat16), tt.astype(jnp.bfloat16))

    u01 = proj[0:2, :, 0].T
    tb = proj[2:].transpose(1, 0, 2).reshape(_H, _K).astype(jnp.bfloat16)
    w2b = W2.astype(jnp.bfloat16)

    nblk = n // _R
    logits = pl.pallas_call(
        _ffn_kernel,
        grid=(nblk,),
        in_specs=[
            pl.BlockSpec((_NT, _R), lambda i: (0, i)),
            pl.BlockSpec((2, _R), lambda i: (0, i)),
            pl.BlockSpec((_H, _K), lambda i: (0, 0)),
            pl.BlockSpec((_H, 2), lambda i: (0, 0)),
            pl.BlockSpec((_H, 1), lambda i: (0, 0)),
            pl.BlockSpec((1, _H), lambda i: (0, 0)),
        ],
        out_specs=pl.BlockSpec((1, 1, _R), lambda i: (i, 0, 0)),
        out_shape=jax.ShapeDtypeStruct((nblk, 1, _R), jnp.float32),
        scratch_shapes=[pltpu.VMEM((_K, _R), jnp.bfloat16)],
    )(idx_t, xd_t, tb, u01, b1.reshape(_H, 1), w2b)
    return logits


def kernel(x, tables, W1, b1, W2, b2):
    logits = _logits(x, tables, W1, b1, W2, b2)
    out = pl.pallas_call(
        _softmax_kernel,
        out_shape=jax.ShapeDtypeStruct((_B, _P), jnp.float32),
    )(logits.reshape(_B, _P))

    return out.reshape(_B, _P, 1)


# speedup vs baseline: 6.5117x; 1.0168x over previous
import functools

import jax
import jax.numpy as jnp
from jax.experimental import pallas as pl
from jax.experimental.pallas import tpu as pltpu

_B, _P, _E, _V, _NT = 1024, 20, 64, 100, 33
_NF = _NT + 2
_VP = 112
_R = 2560
_H = 256
_K = _NT * _VP


def _proj_kernel(w_ref, t_ref, o_ref):
    o_ref[0] = jax.lax.dot(
        w_ref[0], t_ref[0],
        precision=jax.lax.Precision.DEFAULT,
        preferred_element_type=jnp.float32,
    )


def _ffn_kernel(idx_ref, xd_ref, tb_ref, u_ref, b1_ref, w2_ref, o_ref, oh_ref):
    sub = jax.lax.broadcasted_iota(jnp.int32, (_VP, _R), 0)
    for f in range(_NT):
        oh = (sub == idx_ref[f : f + 1, :]).astype(jnp.bfloat16)
        oh_ref[f * _VP : (f + 1) * _VP, :] = oh
    acc = jax.lax.dot(
        tb_ref[...], oh_ref[...],
        precision=jax.lax.Precision.DEFAULT,
        preferred_element_type=jnp.float32,
    )
    acc = acc + u_ref[:, 0:1] * xd_ref[0:1, :] + u_ref[:, 1:2] * xd_ref[1:2, :]
    h = jnp.maximum(acc + b1_ref[...], 0.0)
    o_ref[0] = jax.lax.dot(
        w2_ref[...], h.astype(jnp.bfloat16),
        precision=jax.lax.Precision.DEFAULT,
        preferred_element_type=jnp.float32,
    )


def _softmax_kernel(l_ref, o_ref):
    l = l_ref[...]
    m = jnp.max(l, axis=1, keepdims=True)
    e = jnp.exp(l - m)
    o_ref[...] = e / jnp.sum(e, axis=1, keepdims=True)


def _logits(x, tables, W1, b1, W2, b2):
    n = _B * _P
    idx_t = x[:, :, 2:].astype(jnp.int32).reshape(n, _NT).T
    xd_t = x[:, :, :2].reshape(n, 2).T

    tt = jnp.zeros((_NF, _E, _VP), jnp.float32)
    tt = tt.at[0:2].set(1.0)
    tt = tt.at[2:, :, :_V].set(tables.transpose(0, 2, 1))
    w1t = W1.reshape(_H, _NF, _E).transpose(1, 0, 2)

    proj = pl.pallas_call(
        _proj_kernel,
        grid=(_NF,),
        in_specs=[
            pl.BlockSpec((1, _H, _E), lambda f: (f, 0, 0)),
            pl.BlockSpec((1, _E, _VP), lambda f: (f, 0, 0)),
        ],
        out_specs=pl.BlockSpec((1, _H, _VP), lambda f: (f, 0, 0)),
        out_shape=jax.ShapeDtypeStruct((_NF, _H, _VP), jnp.float32),
    )(w1t.astype(jnp.bfloat16), tt.astype(jnp.bfloat16))

    u01 = proj[0:2, :, 0].T
    tb = proj[2:].transpose(1, 0, 2).reshape(_H, _K).astype(jnp.bfloat16)
    w2b = W2.astype(jnp.bfloat16)

    nblk = n // _R
    logits = pl.pallas_call(
        _ffn_kernel,
        grid=(nblk,),
        in_specs=[
            pl.BlockSpec((_NT, _R), lambda i: (0, i)),
            pl.BlockSpec((2, _R), lambda i: (0, i)),
            pl.BlockSpec((_H, _K), lambda i: (0, 0)),
            pl.BlockSpec((_H, 2), lambda i: (0, 0)),
            pl.BlockSpec((_H, 1), lambda i: (0, 0)),
            pl.BlockSpec((1, _H), lambda i: (0, 0)),
        ],
        out_specs=pl.BlockSpec((1, 1, _R), lambda i: (i, 0, 0)),
        out_shape=jax.ShapeDtypeStruct((nblk, 1, _R), jnp.float32),
        scratch_shapes=[pltpu.VMEM((_K, _R), jnp.bfloat16)],
    )(idx_t, xd_t, tb, u01, b1.reshape(_H, 1), w2b)
    return logits


def kernel(x, tables, W1, b1, W2, b2):
    logits = _logits(x, tables, W1, b1, W2, b2)
    out = pl.pallas_call(
        _softmax_kernel,
        out_shape=jax.ShapeDtypeStruct((_B, _P), jnp.float32),
    )(logits.reshape(_B, _P))

    return out.reshape(_B, _P, 1)
